# baseline (speedup 1.0000x reference)
import functools

import jax
import jax.numpy as jnp
from jax import lax
from jax.experimental import pallas as pl
from jax.experimental.pallas import tpu as pltpu
from jax.experimental.pallas import tpu_sc as plsc

NE = 64
K = 8
D = 2048
T = 16384
R = 512
GRID = T // R
ALPHA = 0.001

NC = 2
NS = 16
NW = NC * NS
L = 16
RW = T // NW
NG = RW // L

_AUX_SCALE = ALPHA * NE / (float(T) * float(K) * float(T))


def _logits_body(x_ref, w_ref, lg_ref, pib_ref, pi_ref):
    step = pl.program_id(0)
    logits = lax.dot_general(
        w_ref[...], x_ref[...], (((1,), (1,)), ((), ())),
        preferred_element_type=jnp.float32)
    lg_ref[...] = logits
    m = jnp.max(logits, axis=0, keepdims=True)
    e = jnp.exp(logits - m)
    s = jnp.sum(e, axis=0, keepdims=True)
    scores = e / s
    pi_part = jnp.sum(scores.reshape(NE, R // 128, 128), axis=1)

    @pl.when(step == 0)
    def _():
        pi_ref[...] = jnp.zeros_like(pi_ref)

    pi_ref[...] += pi_part

    @pl.when(step == GRID - 1)
    def _():
        pib_ref[...] = jnp.broadcast_to(
            jnp.sum(pi_ref[...], axis=1, keepdims=True), (NE, 128))


def _logits_call(xf, weight):
    return pl.pallas_call(
        _logits_body,
        grid=(GRID,),
        in_specs=[
            pl.BlockSpec((R, D), lambda i: (i, 0)),
            pl.BlockSpec((NE, D), lambda i: (0, 0)),
        ],
        out_specs=[
            pl.BlockSpec((NE, R), lambda i: (0, i)),
            pl.BlockSpec((NE, 128), lambda i: (0, 0)),
        ],
        out_shape=[
            jax.ShapeDtypeStruct((NE, T), jnp.float32),
            jax.ShapeDtypeStruct((NE, 128), jnp.float32),
        ],
        scratch_shapes=[
            pltpu.VMEM((NE, 128), jnp.float32),
        ],
        compiler_params=pltpu.CompilerParams(
            dimension_semantics=("arbitrary",)),
    )(xf, weight)


@functools.partial(
    pl.kernel,
    out_type=(
        jax.ShapeDtypeStruct((K, T), jnp.float32),
        jax.ShapeDtypeStruct((K, T), jnp.int32),
        jax.ShapeDtypeStruct((NW, L), jnp.float32),
    ),
    mesh=plsc.VectorSubcoreMesh(core_axis_name="c", subcore_axis_name="s"),
    scratch_types=[
        pltpu.VMEM((NE, RW), jnp.float32),
        pltpu.VMEM((NE, 128), jnp.float32),
        pltpu.VMEM((K, RW), jnp.float32),
        pltpu.VMEM((K, RW), jnp.int32),
        pltpu.VMEM((L,), jnp.float32),
    ],
    compiler_params=pltpu.CompilerParams(needs_layout_passes=False),
)
def _sc_route(lg_hbm, pi_hbm, tw_hbm, ti_hbm, aux_hbm, lv, pv, twv, tiv, av):
    wid = lax.axis_index("s") * NC + lax.axis_index("c")
    base = wid * RW
    pltpu.sync_copy(lg_hbm.at[:, pl.ds(base, RW)], lv)
    pltpu.sync_copy(pi_hbm, pv)

    zero16 = jnp.zeros((L,), jnp.int32)
    lane = lax.iota(jnp.int32, L)
    neg_inf = jnp.full((L,), -jnp.inf, jnp.float32)

    def group_body(g, aux_acc):
        col = g * L

        def expert_body(j, carry):
            vals = list(carry[:K])
            idxs = list(carry[K:])
            c = lv[j, pl.ds(col, L)]
            ci = jnp.broadcast_to(j, (L,)).astype(jnp.int32)
            for k in range(K):
                gt = c > vals[k]
                nv = jnp.where(gt, c, vals[k])
                c = jnp.where(gt, vals[k], c)
                ni = jnp.where(gt, ci, idxs[k])
                ci = jnp.where(gt, idxs[k], ci)
                vals[k] = nv
                idxs[k] = ni
            return tuple(vals) + tuple(idxs)

        init = tuple([neg_inf] * K) + tuple([zero16] * K)
        carry = lax.fori_loop(0, NE, expert_body, init)
        vals = carry[:K]
        idxs = carry[K:]

        es = [jnp.exp(v - vals[0]) for v in vals]
        ssum = es[0]
        for k in range(1, K):
            ssum = ssum + es[k]
        acc = aux_acc
        for k in range(K):
            twv[k, pl.ds(col, L)] = es[k] / ssum
            tiv[k, pl.ds(col, L)] = idxs[k]
            acc = acc + plsc.load_gather(pv, [idxs[k], zero16])
        return acc

    aux_acc = lax.fori_loop(0, NG, group_body, jnp.zeros((L,), jnp.float32))
    av[...] = aux_acc
    pltpu.sync_copy(twv, tw_hbm.at[:, pl.ds(base, RW)])
    pltpu.sync_copy(tiv, ti_hbm.at[:, pl.ds(base, RW)])
    pltpu.sync_copy(av, aux_hbm.at[wid])


def kernel(x, weight):
    xf = x.reshape(T, D)
    lg, pib = _logits_call(xf, weight)
    tw, ti, auxp = _sc_route(lg, pib)
    aux = jnp.sum(auxp) * jnp.float32(_AUX_SCALE)
    return tw.T, ti.T, aux
